# baseline (speedup 1.0000x reference)
import functools

import jax
import jax.numpy as jnp
from jax import lax
from jax.experimental import pallas as pl
from jax.experimental.pallas import tpu as pltpu
from jax.experimental.pallas import tpu_sc as plsc

B = 128
N = 8192
M = 8192
L = 16
NSUB = 16
R = B // NSUB

_mesh = plsc.VectorSubcoreMesh(core_axis_name="c", subcore_axis_name="s",
                               num_cores=2, num_subcores=NSUB)
_params = pltpu.CompilerParams(needs_layout_passes=False)


def _tc_argmax_body(rec_ref, sub_ref, ri_ref, si_ref):
    def row_argmax(x):
        m = jnp.max(x, axis=1, keepdims=True)
        iota = lax.broadcasted_iota(jnp.int32, x.shape, 1)
        return jnp.min(jnp.where(x == m, iota, jnp.int32(2**30)), axis=1)

    ri_ref[...] = row_argmax(rec_ref[...])
    si_ref[...] = row_argmax(sub_ref[...])


_tc_argmax = pl.pallas_call(
    _tc_argmax_body,
    out_shape=(
        jax.ShapeDtypeStruct((B,), jnp.int32),
        jax.ShapeDtypeStruct((B,), jnp.int32),
    ),
)


def _gather_social_body(ri_hbm, si_hbm, vid_hbm, pref_hbm, struct_hbm,
                        tp_hbm, ts_hbm, th_hbm,
                        rb, siv, piv, idxb, vidall, prefrows, rowpv,
                        thw, pib, tpb, thb, tsb, pi_sh,
                        sem0, sem1, sem2, sem3):
    c = lax.axis_index("c")
    s = lax.axis_index("s")
    lane = lax.iota(jnp.int32, L)
    base = s * R
    m8 = lane < R

    @pl.when(c == 0)
    def _():
        cp_r = pltpu.async_copy(ri_hbm.at[pl.ds(base, R)],
                                rb.at[pl.ds(0, R)], sem0)
        cp_s = pltpu.async_copy(si_hbm, siv.at[pl.ds(0, B)], sem1)
        cp_vd = pltpu.async_copy(vid_hbm, vidall, sem2)
        cp_r.wait()
        cp_s.wait()
        myri = rb[...]
        mysi = siv[pl.ds(base, L)]
        cp_pi = pltpu.async_copy(pref_hbm.at[siv.at[pl.ds(base, R)]],
                                 prefrows.at[pl.ds(0, R)], sem1)
        th_copies = []
        for j in range(R):
            col0 = (myri[j] // L) * L
            cp = pltpu.async_copy(
                struct_hbm.at[mysi[j], pl.ds(col0, L)], thw.at[j], sem3)
            th_copies.append(cp)
        cp_vd.wait()
        vid16 = plsc.load_gather(vidall, [jnp.where(m8, myri + 100, 0)])
        idxb[...] = jnp.where(m8, vid16, 0)
        cp_pv = pltpu.async_copy(pref_hbm.at[idxb.at[pl.ds(0, R)]],
                                 rowpv.at[pl.ds(0, R)], sem2)
        cp_pi.wait()
        rowidx = jnp.where(m8, lane, 0)
        colidx = jnp.where(m8, base + lane, 0)
        pi16 = plsc.load_gather(prefrows, [rowidx, colidx])
        pib[...] = pi16
        pltpu.sync_copy(pib.at[pl.ds(0, R)], pi_sh.at[pl.ds(base, R)])
        cp_pv.wait()
        for cp in th_copies:
            cp.wait()
        pv16 = plsc.load_gather(rowpv, [rowidx, colidx])
        th16 = plsc.load_gather(thw, [rowidx, jnp.where(m8, myri % L, 0)])
        tpb[...] = pi16 - pv16
        thb[...] = th16
        pltpu.sync_copy(tpb.at[pl.ds(0, R)], tp_hbm.at[pl.ds(base, R)])
        pltpu.sync_copy(thb.at[pl.ds(0, R)], th_hbm.at[pl.ds(base, R)])

    plsc.subcore_barrier()

    @pl.when(c == 0)
    def _():
        pltpu.sync_copy(pi_sh, piv.at[pl.ds(0, B)])
        ts_vec = jnp.zeros((L,), jnp.float32)
        for j in range(R):
            i = base + j
            ivec = jnp.full((L,), i, jnp.int32)
            s_ib = plsc.load_gather(siv, [ivec])
            p_ib = plsc.load_gather(piv, [ivec])
            acc = jnp.zeros((L,), jnp.float32)
            for cc in range(B // L):
                sl = pl.ds(cc * L, L)
                sik = siv[sl]
                pik = piv[sl]
                idxv = lane + cc * L
                msk = jnp.logical_and(sik == s_ib, idxv != i)
                acc = acc + jnp.where(msk, jnp.minimum(pik, p_ib), 0.0)
            ts_j = jnp.sum(acc)
            ts_vec = jnp.where(lane == j, jnp.full((L,), ts_j, jnp.float32),
                               ts_vec)
        tsb[...] = ts_vec
        pltpu.sync_copy(tsb.at[pl.ds(0, R)], ts_hbm.at[pl.ds(base, R)])


_gather_social_call = functools.partial(
    pl.kernel,
    out_type=(
        jax.ShapeDtypeStruct((B,), jnp.float32),
        jax.ShapeDtypeStruct((B,), jnp.float32),
        jax.ShapeDtypeStruct((B,), jnp.float32),
    ),
    mesh=_mesh,
    scratch_types=[
        pltpu.VMEM((L,), jnp.int32),
        pltpu.VMEM((B + L,), jnp.int32),
        pltpu.VMEM((B + L,), jnp.float32),
        pltpu.VMEM((L,), jnp.int32),
        pltpu.VMEM((100 + N,), jnp.int32),
        pltpu.VMEM((R, B), jnp.float32),
        pltpu.VMEM((R, B), jnp.float32),
        pltpu.VMEM((R, L), jnp.float32),
        pltpu.VMEM((L,), jnp.float32),
        pltpu.VMEM((L,), jnp.float32),
        pltpu.VMEM((L,), jnp.float32),
        pltpu.VMEM((L,), jnp.float32),
        pltpu.VMEM_SHARED((B,), jnp.float32),
        pltpu.SemaphoreType.DMA,
        pltpu.SemaphoreType.DMA,
        pltpu.SemaphoreType.DMA,
        pltpu.SemaphoreType.DMA,
    ],
    compiler_params=_params,
)(_gather_social_body)


def kernel(Recommended_m, Substitute_m, ItemGroups_m, Vid, VUU, KUU, Vscore,
           Kscore, preference, structure):
    del ItemGroups_m, VUU, KUU, Vscore, Kscore
    ri, si = _tc_argmax(Recommended_m, Substitute_m)
    tp, ts, th = _gather_social_call(ri, si, Vid, preference, structure)
    return tp[:, None], ts[:, None], th[:, None]
